# baseline (speedup 1.0000x reference)
import functools

import jax
import jax.numpy as jnp
from jax import lax
from jax.experimental import pallas as pl
from jax.experimental.pallas import tpu as pltpu
from jax.experimental.pallas import tpu_sc as plsc

B = 8
H = 224
W = 224
HW = H * W
N_M = 4096
N_UP = 1024
N_UT = 512
NMH = N_M // 2
NUPH = N_UP // 2
NUTH = N_UT // 2
LANES = 16
MB_IT = NMH // LANES
UP_IT = NUPH // LANES
UT_IT = NUTH // LANES


def _sc_loss_kernel(input_hbm, target_hbm, pmb_hbm, pmd_hbm, tmb_hbm,
                    tmd_hbm, pub_hbm, pud_hbm, tub_hbm, tud_hbm, out_hbm,
                    cmb_v, cmd_v, cub_v, cud_v, field_v, valb_v, vald_v,
                    pairb_v, paird_v, part_v, red_v, out_v,
                    matched_sp, partials_sp):
    c = lax.axis_index("c")
    s = lax.axis_index("s")
    f = s % 2
    lb = (s // 2) % 4
    h = s // 8
    b = c * 4 + lb

    is_pred = f == 0

    @pl.when(is_pred)
    def _stage_pred():
        pltpu.sync_copy(input_hbm.at[pl.ds(b * HW, HW)], field_v)
        mo = b * 2 * N_M + h * 2 * NMH
        pltpu.sync_copy(pmb_hbm.at[pl.ds(mo, 2 * NMH)], cmb_v)
        pltpu.sync_copy(pmd_hbm.at[pl.ds(mo, 2 * NMH)], cmd_v)
        uo = b * 2 * N_UP + h * 2 * NUPH
        pltpu.sync_copy(pub_hbm.at[pl.ds(uo, 2 * NUPH)], cub_v)
        pltpu.sync_copy(pud_hbm.at[pl.ds(uo, 2 * NUPH)], cud_v)

    @pl.when(jnp.logical_not(is_pred))
    def _stage_tgt():
        pltpu.sync_copy(target_hbm.at[pl.ds(b * HW, HW)], field_v)
        mo = b * 2 * N_M + h * 2 * NMH
        pltpu.sync_copy(tmb_hbm.at[pl.ds(mo, 2 * NMH)], cmb_v)
        pltpu.sync_copy(tmd_hbm.at[pl.ds(mo, 2 * NMH)], cmd_v)
        uo = b * 2 * N_UT + h * 2 * NUTH
        pltpu.sync_copy(tub_hbm.at[pl.ds(uo, 2 * NUTH)], cub_v.at[pl.ds(0, 2 * NUTH)])
        pltpu.sync_copy(tud_hbm.at[pl.ds(uo, 2 * NUTH)], cud_v.at[pl.ds(0, 2 * NUTH)])

    lane2 = jnp.arange(16, dtype=jnp.int32) * 2

    def activate(v):
        return jnp.where(is_pred, 1.0 / (1.0 + jnp.exp(-v)), v)

    def pair_gather(cbuf, off):
        rs = plsc.load_gather(cbuf, [off + lane2])
        cs = plsc.load_gather(cbuf, [off + lane2 + 1])
        return plsc.load_gather(field_v, [rs * W + cs])

    def matched_body(i, _):
        o = i * (2 * LANES)
        valb_v[pl.ds(i * LANES, LANES)] = activate(pair_gather(cmb_v, o))
        vald_v[pl.ds(i * LANES, LANES)] = activate(pair_gather(cmd_v, o))
        return 0
    lax.fori_loop(0, MB_IT, matched_body, 0, unroll=2)

    def unmatched_body(i, acc):
        o = i * (2 * LANES)
        d = activate(pair_gather(cub_v, o)) - activate(pair_gather(cud_v, o))
        return acc + d * d
    n_u = jnp.where(is_pred, UP_IT, UT_IT)
    acc = lax.fori_loop(0, n_u, unmatched_body,
                        jnp.zeros((LANES,), jnp.float32))
    part_v[...] = acc

    @pl.when(jnp.logical_not(is_pred))
    def _publish():
        pltpu.sync_copy(valb_v, matched_sp.at[lb, 0, pl.ds(h * NMH, NMH)])
        pltpu.sync_copy(vald_v, matched_sp.at[lb, 1, pl.ds(h * NMH, NMH)])

    plsc.subcore_barrier()

    @pl.when(is_pred)
    def _matched_loss():
        pltpu.sync_copy(matched_sp.at[lb, 0, pl.ds(h * NMH, NMH)], pairb_v)
        pltpu.sync_copy(matched_sp.at[lb, 1, pl.ds(h * NMH, NMH)], paird_v)

        def body(i, acc):
            o = i * LANES
            db = valb_v[pl.ds(o, LANES)] - pairb_v[pl.ds(o, LANES)]
            dd = vald_v[pl.ds(o, LANES)] - paird_v[pl.ds(o, LANES)]
            return acc + 2.0 * (db * db + dd * dd)
        m_acc = lax.fori_loop(0, MB_IT, body,
                              jnp.zeros((LANES,), jnp.float32), unroll=2)
        part_v[...] = part_v[...] + m_acc

    pltpu.sync_copy(part_v, partials_sp.at[pl.ds(s * LANES, LANES)])
    plsc.subcore_barrier()

    @pl.when(s == 0)
    def _reduce():
        pltpu.sync_copy(partials_sp, red_v)

        def body(j, acc):
            return acc + red_v[pl.ds(j * LANES, LANES)]
        tot = lax.fori_loop(0, 16, body, jnp.zeros((LANES,), jnp.float32))
        total = jnp.sum(tot) * jnp.float32(1.0 / B)
        out_v[...] = jnp.broadcast_to(total, (LANES,))
        pltpu.sync_copy(out_v, out_hbm.at[c])


_sc_loss = functools.partial(
    pl.kernel,
    mesh=plsc.VectorSubcoreMesh(core_axis_name="c", subcore_axis_name="s"),
    out_type=jax.ShapeDtypeStruct((2, LANES), jnp.float32),
    compiler_params=pltpu.CompilerParams(needs_layout_passes=False),
    scratch_types=[
        pltpu.VMEM((2 * NMH,), jnp.int32),
        pltpu.VMEM((2 * NMH,), jnp.int32),
        pltpu.VMEM((2 * NUPH,), jnp.int32),
        pltpu.VMEM((2 * NUPH,), jnp.int32),
        pltpu.VMEM((HW,), jnp.float32),
        pltpu.VMEM((NMH,), jnp.float32),
        pltpu.VMEM((NMH,), jnp.float32),
        pltpu.VMEM((NMH,), jnp.float32),
        pltpu.VMEM((NMH,), jnp.float32),
        pltpu.VMEM((LANES,), jnp.float32),
        pltpu.VMEM((16 * LANES,), jnp.float32),
        pltpu.VMEM((LANES,), jnp.float32),
        pltpu.VMEM_SHARED((4, 2, N_M), jnp.float32),
        pltpu.VMEM_SHARED((16 * LANES,), jnp.float32),
    ],
)(_sc_loss_kernel)


@jax.jit
def kernel(input, target, pred_mb, pred_md, tgt_mb, tgt_md,
           pred_ub, pred_ud, tgt_ub, tgt_ud):
    def flat(x):
        return x.astype(jnp.int32).reshape(-1)

    out = _sc_loss(input.reshape(-1), target.reshape(-1),
                   flat(pred_mb), flat(pred_md), flat(tgt_mb), flat(tgt_md),
                   flat(pred_ub), flat(pred_ud), flat(tgt_ub), flat(tgt_ud))
    return out[0, 0] + out[1, 0]


# speedup vs baseline: 2.5589x; 2.5589x over previous
import functools

import jax
import jax.numpy as jnp
from jax import lax
from jax.experimental import pallas as pl
from jax.experimental.pallas import tpu as pltpu
from jax.experimental.pallas import tpu_sc as plsc

B = 8
H = 224
W = 224
HW = H * W
N_M = 4096
N_UP = 1024
N_UT = 512
NMH = N_M // 2
NUPH = N_UP // 2
NUTH = N_UT // 2
LANES = 16
MB_IT = NMH // LANES
UP_IT = NUPH // LANES
UT_IT = NUTH // LANES

O_PMB = 0
O_PMD = O_PMB + B * N_M
O_PUB = O_PMD + B * N_M
O_PUD = O_PUB + B * N_UP
O_TMB = O_PUD + B * N_UP
O_TMD = O_TMB + B * N_M
O_TUB = O_TMD + B * N_M
O_TUD = O_TUB + B * N_UT
N_IDX = O_TUD + B * N_UT


def _sc_loss_kernel(input_hbm, target_hbm, rows_hbm, cols_hbm, out_hbm,
                    mbr_v, mbc_v, mdr_v, mdc_v, ubr_v, ubc_v, udr_v, udc_v,
                    field_v, valb_v, vald_v, pairb_v, paird_v,
                    part_v, red_v, out_v, matched_sp, partials_sp):
    c = lax.axis_index("c")
    s = lax.axis_index("s")
    f = s % 2
    lb = (s // 2) % 4
    h = s // 8
    b = c * 4 + lb

    is_pred = f == 0

    @pl.when(is_pred)
    def _stage_pred():
        pltpu.sync_copy(input_hbm.at[pl.ds(b * HW, HW)], field_v)
        mo = b * N_M + h * NMH
        pltpu.sync_copy(rows_hbm.at[pl.ds(O_PMB + mo, NMH)], mbr_v)
        pltpu.sync_copy(cols_hbm.at[pl.ds(O_PMB + mo, NMH)], mbc_v)
        pltpu.sync_copy(rows_hbm.at[pl.ds(O_PMD + mo, NMH)], mdr_v)
        pltpu.sync_copy(cols_hbm.at[pl.ds(O_PMD + mo, NMH)], mdc_v)
        uo = b * N_UP + h * NUPH
        pltpu.sync_copy(rows_hbm.at[pl.ds(O_PUB + uo, NUPH)], ubr_v)
        pltpu.sync_copy(cols_hbm.at[pl.ds(O_PUB + uo, NUPH)], ubc_v)
        pltpu.sync_copy(rows_hbm.at[pl.ds(O_PUD + uo, NUPH)], udr_v)
        pltpu.sync_copy(cols_hbm.at[pl.ds(O_PUD + uo, NUPH)], udc_v)

    @pl.when(jnp.logical_not(is_pred))
    def _stage_tgt():
        pltpu.sync_copy(target_hbm.at[pl.ds(b * HW, HW)], field_v)
        mo = b * N_M + h * NMH
        pltpu.sync_copy(rows_hbm.at[pl.ds(O_TMB + mo, NMH)], mbr_v)
        pltpu.sync_copy(cols_hbm.at[pl.ds(O_TMB + mo, NMH)], mbc_v)
        pltpu.sync_copy(rows_hbm.at[pl.ds(O_TMD + mo, NMH)], mdr_v)
        pltpu.sync_copy(cols_hbm.at[pl.ds(O_TMD + mo, NMH)], mdc_v)
        uo = b * N_UT + h * NUTH
        pltpu.sync_copy(rows_hbm.at[pl.ds(O_TUB + uo, NUTH)],
                        ubr_v.at[pl.ds(0, NUTH)])
        pltpu.sync_copy(cols_hbm.at[pl.ds(O_TUB + uo, NUTH)],
                        ubc_v.at[pl.ds(0, NUTH)])
        pltpu.sync_copy(rows_hbm.at[pl.ds(O_TUD + uo, NUTH)],
                        udr_v.at[pl.ds(0, NUTH)])
        pltpu.sync_copy(cols_hbm.at[pl.ds(O_TUD + uo, NUTH)],
                        udc_v.at[pl.ds(0, NUTH)])

    def activate(v):
        return jnp.where(is_pred, 1.0 / (1.0 + jnp.exp(-v)), v)

    def pair_gather(rbuf, cbuf, off):
        r = rbuf[pl.ds(off, LANES)]
        col = cbuf[pl.ds(off, LANES)]
        return plsc.load_gather(field_v, [r * W + col])

    def matched_body(i, _):
        o = i * LANES
        valb_v[pl.ds(o, LANES)] = activate(pair_gather(mbr_v, mbc_v, o))
        vald_v[pl.ds(o, LANES)] = activate(pair_gather(mdr_v, mdc_v, o))
        return 0
    lax.fori_loop(0, MB_IT, matched_body, 0, unroll=2)

    def unmatched_body(i, acc):
        o = i * LANES
        d = (activate(pair_gather(ubr_v, ubc_v, o))
             - activate(pair_gather(udr_v, udc_v, o)))
        return acc + d * d
    n_u = jnp.where(is_pred, UP_IT, UT_IT)
    acc = lax.fori_loop(0, n_u, unmatched_body,
                        jnp.zeros((LANES,), jnp.float32))
    part_v[...] = acc

    @pl.when(jnp.logical_not(is_pred))
    def _publish():
        pltpu.sync_copy(valb_v, matched_sp.at[lb, 0, pl.ds(h * NMH, NMH)])
        pltpu.sync_copy(vald_v, matched_sp.at[lb, 1, pl.ds(h * NMH, NMH)])

    plsc.subcore_barrier()

    @pl.when(is_pred)
    def _matched_loss():
        pltpu.sync_copy(matched_sp.at[lb, 0, pl.ds(h * NMH, NMH)], pairb_v)
        pltpu.sync_copy(matched_sp.at[lb, 1, pl.ds(h * NMH, NMH)], paird_v)

        def body(i, acc):
            o = i * LANES
            db = valb_v[pl.ds(o, LANES)] - pairb_v[pl.ds(o, LANES)]
            dd = vald_v[pl.ds(o, LANES)] - paird_v[pl.ds(o, LANES)]
            return acc + 2.0 * (db * db + dd * dd)
        m_acc = lax.fori_loop(0, MB_IT, body,
                              jnp.zeros((LANES,), jnp.float32), unroll=2)
        part_v[...] = part_v[...] + m_acc

    pltpu.sync_copy(part_v, partials_sp.at[pl.ds(s * LANES, LANES)])
    plsc.subcore_barrier()

    @pl.when(s == 0)
    def _reduce():
        pltpu.sync_copy(partials_sp, red_v)

        def body(j, acc):
            return acc + red_v[pl.ds(j * LANES, LANES)]
        tot = lax.fori_loop(0, 16, body, jnp.zeros((LANES,), jnp.float32))
        total = jnp.sum(tot) * jnp.float32(1.0 / B)
        out_v[...] = jnp.broadcast_to(total, (LANES,))
        pltpu.sync_copy(out_v, out_hbm.at[c])


_sc_loss = functools.partial(
    pl.kernel,
    mesh=plsc.VectorSubcoreMesh(core_axis_name="c", subcore_axis_name="s"),
    out_type=jax.ShapeDtypeStruct((2, LANES), jnp.float32),
    compiler_params=pltpu.CompilerParams(needs_layout_passes=False),
    scratch_types=[
        pltpu.VMEM((NMH,), jnp.int32),
        pltpu.VMEM((NMH,), jnp.int32),
        pltpu.VMEM((NMH,), jnp.int32),
        pltpu.VMEM((NMH,), jnp.int32),
        pltpu.VMEM((NUPH,), jnp.int32),
        pltpu.VMEM((NUPH,), jnp.int32),
        pltpu.VMEM((NUPH,), jnp.int32),
        pltpu.VMEM((NUPH,), jnp.int32),
        pltpu.VMEM((HW,), jnp.float32),
        pltpu.VMEM((NMH,), jnp.float32),
        pltpu.VMEM((NMH,), jnp.float32),
        pltpu.VMEM((NMH,), jnp.float32),
        pltpu.VMEM((NMH,), jnp.float32),
        pltpu.VMEM((LANES,), jnp.float32),
        pltpu.VMEM((16 * LANES,), jnp.float32),
        pltpu.VMEM((LANES,), jnp.float32),
        pltpu.VMEM_SHARED((4, 2, N_M), jnp.float32),
        pltpu.VMEM_SHARED((16 * LANES,), jnp.float32),
    ],
)(_sc_loss_kernel)


@jax.jit
def kernel(input, target, pred_mb, pred_md, tgt_mb, tgt_md,
           pred_ub, pred_ud, tgt_ub, tgt_ud):
    coords = (pred_mb, pred_md, pred_ub, pred_ud,
              tgt_mb, tgt_md, tgt_ub, tgt_ud)

    def comp(k):
        return jnp.concatenate(
            [x[..., k].astype(jnp.int32).reshape(-1) for x in coords])

    out = _sc_loss(input.reshape(-1), target.reshape(-1), comp(0), comp(1))
    return out[0, 0] + out[1, 0]


# speedup vs baseline: 3.2786x; 1.2813x over previous
import functools

import jax
import jax.numpy as jnp
from jax import lax
from jax.experimental import pallas as pl
from jax.experimental.pallas import tpu as pltpu
from jax.experimental.pallas import tpu_sc as plsc

B = 8
H = 224
W = 224
HW = H * W
N_M = 4096
N_UP = 1024
N_UT = 512
LANES = 16
NMQ = N_M // 4
NUPQ = N_UP // 4
NUTQ = N_UT // 4
O_PMD = NMQ
O_TMB = 2 * NMQ
O_TMD = 3 * NMQ
O_PUB = 4 * NMQ
O_PUD = O_PUB + NUPQ
O_TUB = O_PUD + NUPQ
O_TUD = O_TUB + NUTQ
PER_TILE = O_TUD + NUTQ
PER_IMG = 4 * PER_TILE
M_IT = NMQ // LANES
UP_IT = NUPQ // LANES
UT_IT = NUTQ // LANES


def _sc_loss_kernel(input_hbm, target_hbm, idx_hbm, out_hbm,
                    idx_v, fp_v, ft_v, part_v, red_v, out_v,
                    sem0, sem1, sem2, partials_sp):
    c = lax.axis_index("c")
    s = lax.axis_index("s")
    q = s % 4
    b = c * 4 + s // 4

    cp0 = pltpu.async_copy(idx_hbm.at[pl.ds(b * PER_IMG + q * PER_TILE,
                                            PER_TILE)], idx_v, sem0)
    cp1 = pltpu.async_copy(input_hbm.at[pl.ds(b * HW, HW)], fp_v, sem1)
    cp2 = pltpu.async_copy(target_hbm.at[pl.ds(b * HW, HW)], ft_v, sem2)
    cp0.wait()
    cp1.wait()
    cp2.wait()

    def sig(v):
        return 1.0 / (1.0 + jnp.exp(-v))

    def g(field, off):
        return plsc.load_gather(field, [idx_v[pl.ds(off, LANES)]])

    def matched_body(i, acc):
        o = i * LANES
        db = sig(g(fp_v, o)) - g(ft_v, O_TMB + o)
        dd = sig(g(fp_v, O_PMD + o)) - g(ft_v, O_TMD + o)
        return acc + (db * db + dd * dd)
    acc = lax.fori_loop(0, M_IT, matched_body,
                        jnp.zeros((LANES,), jnp.float32), unroll=2)
    acc = acc + acc

    def up_body(i, acc):
        o = i * LANES
        d = sig(g(fp_v, O_PUB + o)) - sig(g(fp_v, O_PUD + o))
        return acc + d * d
    acc = lax.fori_loop(0, UP_IT, up_body, acc, unroll=2)

    def ut_body(i, acc):
        o = i * LANES
        d = g(ft_v, O_TUB + o) - g(ft_v, O_TUD + o)
        return acc + d * d
    acc = lax.fori_loop(0, UT_IT, ut_body, acc, unroll=2)

    part_v[...] = acc
    pltpu.sync_copy(part_v, partials_sp.at[pl.ds(s * LANES, LANES)])
    plsc.subcore_barrier()

    @pl.when(s == 0)
    def _reduce():
        pltpu.sync_copy(partials_sp, red_v)

        def body(j, acc):
            return acc + red_v[pl.ds(j * LANES, LANES)]
        tot = lax.fori_loop(0, 16, body, jnp.zeros((LANES,), jnp.float32))
        total = jnp.sum(tot) * jnp.float32(1.0 / B)
        out_v[...] = jnp.broadcast_to(total, (LANES,))
        pltpu.sync_copy(out_v, out_hbm.at[c])


_sc_loss = functools.partial(
    pl.kernel,
    mesh=plsc.VectorSubcoreMesh(core_axis_name="c", subcore_axis_name="s"),
    out_type=jax.ShapeDtypeStruct((2, LANES), jnp.float32),
    compiler_params=pltpu.CompilerParams(needs_layout_passes=False),
    scratch_types=[
        pltpu.VMEM((PER_TILE,), jnp.int32),
        pltpu.VMEM((HW,), jnp.float32),
        pltpu.VMEM((HW,), jnp.float32),
        pltpu.VMEM((LANES,), jnp.float32),
        pltpu.VMEM((16 * LANES,), jnp.float32),
        pltpu.VMEM((LANES,), jnp.float32),
        pltpu.SemaphoreType.DMA,
        pltpu.SemaphoreType.DMA,
        pltpu.SemaphoreType.DMA,
        pltpu.VMEM_SHARED((16 * LANES,), jnp.float32),
    ],
)(_sc_loss_kernel)


@jax.jit
def kernel(input, target, pred_mb, pred_md, tgt_mb, tgt_md,
           pred_ub, pred_ud, tgt_ub, tgt_ud):
    def flat_idx(x):
        x = x.astype(jnp.int32)
        return x[..., 0] * W + x[..., 1]

    pmb, pmd, tmb, tmd = map(flat_idx, (pred_mb, pred_md, tgt_mb, tgt_md))
    pub, pud, tub, tud = map(flat_idx, (pred_ub, pred_ud, tgt_ub, tgt_ud))

    parts = []
    for q in range(4):
        def qs(x, n):
            return x[:, q * n:(q + 1) * n]
        parts += [qs(pmb, NMQ), qs(pmd, NMQ), qs(tmb, NMQ), qs(tmd, NMQ),
                  qs(pub, NUPQ), qs(pud, NUPQ), qs(tub, NUTQ), qs(tud, NUTQ)]
    idx = jnp.concatenate(parts, axis=1).reshape(-1)

    out = _sc_loss(input.reshape(-1), target.reshape(-1), idx)
    return out[0, 0] + out[1, 0]


# speedup vs baseline: 4.1569x; 1.2679x over previous
import functools

import jax
import jax.numpy as jnp
from jax import lax
from jax.experimental import pallas as pl
from jax.experimental.pallas import tpu as pltpu
from jax.experimental.pallas import tpu_sc as plsc

B = 8
H = 224
W = 224
HW = H * W
N_M = 4096
N_UP = 1024
N_UT = 512
LANES = 16
NMQ = N_M // 4
NUPQ = N_UP // 4
NUTQ = N_UT // 4
O_PMB = 0
O_PMD = N_M
O_TMB = 2 * N_M
O_TMD = 3 * N_M
O_PUB = 4 * N_M
O_PUD = O_PUB + N_UP
O_TUB = O_PUD + N_UP
O_TUD = O_TUB + N_UT
N_ALL = O_TUD + N_UT
M_IT = NMQ // LANES
UP_IT = NUPQ // LANES
UT_IT = NUTQ // LANES


def _sc_loss_kernel(input_hbm, target_hbm, coords_hbm, out_hbm,
                    fp_v, ft_v,
                    mbr_v, mbc_v, mdr_v, mdc_v,
                    tbr_v, tbc_v, tdr_v, tdc_v,
                    ubr_v, ubc_v, udr_v, udc_v,
                    vbr_v, vbc_v, vdr_v, vdc_v,
                    part_v, red_v, out_v, sem,
                    partials_sp):
    c = lax.axis_index("c")
    s = lax.axis_index("s")
    q = s % 4
    b = c * 4 + s // 4

    cps = [
        pltpu.async_copy(input_hbm.at[b, 0], fp_v, sem),
        pltpu.async_copy(target_hbm.at[b, 0], ft_v, sem),
    ]
    for off, n, rbuf, cbuf in (
            (O_PMB, NMQ, mbr_v, mbc_v), (O_PMD, NMQ, mdr_v, mdc_v),
            (O_TMB, NMQ, tbr_v, tbc_v), (O_TMD, NMQ, tdr_v, tdc_v),
            (O_PUB, NUPQ, ubr_v, ubc_v), (O_PUD, NUPQ, udr_v, udc_v),
            (O_TUB, NUTQ, vbr_v, vbc_v), (O_TUD, NUTQ, vdr_v, vdc_v)):
        cps.append(pltpu.async_copy(
            coords_hbm.at[b, 0, pl.ds(off + q * n, n)], rbuf, sem))
        cps.append(pltpu.async_copy(
            coords_hbm.at[b, 1, pl.ds(off + q * n, n)], cbuf, sem))
    for cp in cps:
        cp.wait()

    def sig(v):
        return 1.0 / (1.0 + jnp.exp(-v))

    def g(field, rbuf, cbuf, o):
        return plsc.load_gather(
            field, [rbuf[pl.ds(o, LANES)], cbuf[pl.ds(o, LANES)]])

    def matched_body(i, acc):
        o = i * LANES
        db = sig(g(fp_v, mbr_v, mbc_v, o)) - g(ft_v, tbr_v, tbc_v, o)
        dd = sig(g(fp_v, mdr_v, mdc_v, o)) - g(ft_v, tdr_v, tdc_v, o)
        return acc + (db * db + dd * dd)
    acc = lax.fori_loop(0, M_IT, matched_body,
                        jnp.zeros((LANES,), jnp.float32), unroll=2)
    acc = acc + acc

    def up_body(i, acc):
        o = i * LANES
        d = sig(g(fp_v, ubr_v, ubc_v, o)) - sig(g(fp_v, udr_v, udc_v, o))
        return acc + d * d
    acc = lax.fori_loop(0, UP_IT, up_body, acc, unroll=2)

    def ut_body(i, acc):
        o = i * LANES
        d = g(ft_v, vbr_v, vbc_v, o) - g(ft_v, vdr_v, vdc_v, o)
        return acc + d * d
    acc = lax.fori_loop(0, UT_IT, ut_body, acc, unroll=2)

    part_v[...] = acc
    pltpu.sync_copy(part_v, partials_sp.at[pl.ds(s * LANES, LANES)])
    plsc.subcore_barrier()

    @pl.when(s == 0)
    def _reduce():
        pltpu.sync_copy(partials_sp, red_v)

        def body(j, acc):
            return acc + red_v[pl.ds(j * LANES, LANES)]
        tot = lax.fori_loop(0, 16, body, jnp.zeros((LANES,), jnp.float32))
        total = jnp.sum(tot) * jnp.float32(1.0 / B)
        out_v[...] = jnp.broadcast_to(total, (LANES,))
        pltpu.sync_copy(out_v, out_hbm.at[c])


_sc_loss = functools.partial(
    pl.kernel,
    mesh=plsc.VectorSubcoreMesh(core_axis_name="c", subcore_axis_name="s"),
    out_type=jax.ShapeDtypeStruct((2, LANES), jnp.float32),
    compiler_params=pltpu.CompilerParams(needs_layout_passes=False),
    scratch_types=[
        pltpu.VMEM((H, W), jnp.float32),
        pltpu.VMEM((H, W), jnp.float32),
        pltpu.VMEM((NMQ,), jnp.int32),
        pltpu.VMEM((NMQ,), jnp.int32),
        pltpu.VMEM((NMQ,), jnp.int32),
        pltpu.VMEM((NMQ,), jnp.int32),
        pltpu.VMEM((NMQ,), jnp.int32),
        pltpu.VMEM((NMQ,), jnp.int32),
        pltpu.VMEM((NMQ,), jnp.int32),
        pltpu.VMEM((NMQ,), jnp.int32),
        pltpu.VMEM((NUPQ,), jnp.int32),
        pltpu.VMEM((NUPQ,), jnp.int32),
        pltpu.VMEM((NUPQ,), jnp.int32),
        pltpu.VMEM((NUPQ,), jnp.int32),
        pltpu.VMEM((NUTQ,), jnp.int32),
        pltpu.VMEM((NUTQ,), jnp.int32),
        pltpu.VMEM((NUTQ,), jnp.int32),
        pltpu.VMEM((NUTQ,), jnp.int32),
        pltpu.VMEM((LANES,), jnp.float32),
        pltpu.VMEM((16 * LANES,), jnp.float32),
        pltpu.VMEM((LANES,), jnp.float32),
        pltpu.SemaphoreType.DMA,
        pltpu.VMEM_SHARED((16 * LANES,), jnp.float32),
    ],
)(_sc_loss_kernel)


@jax.jit
def kernel(input, target, pred_mb, pred_md, tgt_mb, tgt_md,
           pred_ub, pred_ud, tgt_ub, tgt_ud):
    coords = jnp.concatenate(
        [pred_mb, pred_md, tgt_mb, tgt_md,
         pred_ub, pred_ud, tgt_ub, tgt_ud], axis=1).astype(jnp.int32)
    coords_t = coords.transpose(0, 2, 1)
    out = _sc_loss(input, target, coords_t)
    return out[0, 0] + out[1, 0]
